# initial kernel scaffold (unmeasured)
import jax
import jax.numpy as jnp
from jax import lax
from jax.experimental import pallas as pl
from jax.experimental.pallas import tpu as pltpu

N_DEV = 8
SLOTS = 4


def _ring(q):
    return jnp.where(q < 4, q, 11 - q)


def kernel(x, w_mat):
    m_full, k = x.shape
    m_half = m_full // 2
    n_per = w_mat.shape[1]
    m_tot = N_DEV * m_full

    def body(x_ref, w_ref, out_ref, cf, cb, amax_tx, amax_rx,
             sf_sems, rf_sems, sb_sems, rb_sems, a_send, a_recv):
        l = lax.axis_index("i")
        p = _ring(l)
        nxt = _ring((p + 1) % N_DEV)
        prv = _ring((p - 1) % N_DEV)

        cf[0, :, :] = x_ref[:m_half, :]
        cb[0, :, :] = x_ref[m_half:, :]

        w = w_ref[...]

        def gemm(chunk, origin_l, bottom):
            blk = jnp.dot(chunk, w, preferred_element_type=jnp.float32,
                          precision=lax.Precision.HIGHEST)
            row0 = origin_l * m_full + (m_half if bottom else 0)
            out_ref[pl.ds(row0, m_half), :] = blk
            return jnp.max(jnp.abs(blk))

        amax = jnp.float32(0)
        for h in range(N_DEV - 1):
            s = h % SLOTS
            r = (h + 1) % SLOTS
            rf = pltpu.make_async_remote_copy(
                src_ref=cf.at[s], dst_ref=cf.at[r],
                send_sem=sf_sems.at[s], recv_sem=rf_sems.at[r],
                device_id=(nxt,), device_id_type=pl.DeviceIdType.MESH)
            rb = pltpu.make_async_remote_copy(
                src_ref=cb.at[s], dst_ref=cb.at[r],
                send_sem=sb_sems.at[s], recv_sem=rb_sems.at[r],
                device_id=(prv,), device_id_type=pl.DeviceIdType.MESH)
            rf.start()
            rb.start()
            of = _ring((p - h) % N_DEV)
            ob = _ring((p + h) % N_DEV)
            amax = jnp.maximum(amax, gemm(cf[s, :, :], of, False))
            amax = jnp.maximum(amax, gemm(cb[s, :, :], ob, True))
            rf.wait()
            rb.wait()
        s_last = (N_DEV - 1) % SLOTS
        amax = jnp.maximum(amax, gemm(cf[s_last, :, :], nxt, False))
        amax = jnp.maximum(amax, gemm(cb[s_last, :, :], prv, True))

        amax_tx[0, :] = jnp.full((128,), amax, jnp.float32)
        amax_rx[pl.ds(l, 1), :] = amax_tx[0:1, :]
        descs = []
        for j in range(1, N_DEV):
            tgt = (l + j) % N_DEV
            d = pltpu.make_async_remote_copy(
                src_ref=amax_tx.at[pl.ds(0, 1)],
                dst_ref=amax_rx.at[pl.ds(l, 1)],
                send_sem=a_send.at[j], recv_sem=a_recv.at[j],
                device_id=(tgt,), device_id_type=pl.DeviceIdType.MESH)
            d.start()
            descs.append(d)
        for d in descs:
            d.wait()

        scale = jnp.max(amax_rx[...]) / 448.0
        y = out_ref[...]
        q = (y / scale).astype(jnp.float8_e4m3fn).astype(jnp.float32)
        out_ref[...] = q * scale

    return pl.pallas_call(
        body,
        out_shape=jax.ShapeDtypeStruct((m_tot, n_per), jnp.float32),
        in_specs=[pl.BlockSpec(memory_space=pltpu.VMEM),
                  pl.BlockSpec(memory_space=pltpu.VMEM)],
        out_specs=pl.BlockSpec(memory_space=pltpu.VMEM),
        scratch_shapes=[
            pltpu.VMEM((SLOTS, m_half, k), jnp.float32),
            pltpu.VMEM((SLOTS, m_half, k), jnp.float32),
            pltpu.VMEM((1, 128), jnp.float32),
            pltpu.VMEM((N_DEV, 128), jnp.float32),
            pltpu.SemaphoreType.DMA((SLOTS,)),
            pltpu.SemaphoreType.DMA((SLOTS,)),
            pltpu.SemaphoreType.DMA((SLOTS,)),
            pltpu.SemaphoreType.DMA((SLOTS,)),
            pltpu.SemaphoreType.DMA((N_DEV,)),
            pltpu.SemaphoreType.DMA((N_DEV,)),
        ],
    )(x, w_mat)


# baseline (device time: 353693 ns/iter reference)
import jax
import jax.numpy as jnp
from jax import lax
from jax.experimental import pallas as pl
from jax.experimental.pallas import tpu as pltpu

N_DEV = 8
SLOTS = 2


def _ring(q):
    return jnp.where(q < 4, q, 11 - q)


def kernel(x, w_mat):
    m_full, k = x.shape
    m_half = m_full // 2
    n_per = w_mat.shape[1]
    m_tot = N_DEV * m_full

    def body(x_ref, w_ref, out_ref, cf, cb, amax_tx, amax_rx,
             sf_sems, rf_sems, sb_sems, rb_sems, a_send, a_recv):
        l = lax.axis_index("i")
        p = _ring(l)
        nxt = _ring((p + 1) % N_DEV)
        prv = _ring((p - 1) % N_DEV)

        cf[0, :, :] = x_ref[:m_half, :]
        cb[0, :, :] = x_ref[m_half:, :]

        w = w_ref[...]

        def gemm(chunk, origin_l, bottom):
            blk = jnp.dot(chunk, w, preferred_element_type=jnp.float32,
                          precision=lax.Precision.HIGHEST)
            row0 = origin_l * m_full + (m_half if bottom else 0)
            out_ref[pl.ds(row0, m_half), :] = blk
            return jnp.max(jnp.abs(blk))

        amax = jnp.float32(0)
        for h in range(N_DEV - 1):
            s = h % SLOTS
            r = (h + 1) % SLOTS
            rf = pltpu.make_async_remote_copy(
                src_ref=cf.at[s], dst_ref=cf.at[r],
                send_sem=sf_sems.at[s], recv_sem=rf_sems.at[r],
                device_id=(nxt,), device_id_type=pl.DeviceIdType.MESH)
            rb = pltpu.make_async_remote_copy(
                src_ref=cb.at[s], dst_ref=cb.at[r],
                send_sem=sb_sems.at[s], recv_sem=rb_sems.at[r],
                device_id=(prv,), device_id_type=pl.DeviceIdType.MESH)
            rf.start()
            rb.start()
            of = _ring((p - h) % N_DEV)
            ob = _ring((p + h) % N_DEV)
            amax = jnp.maximum(amax, gemm(cf[s, :, :], of, False))
            amax = jnp.maximum(amax, gemm(cb[s, :, :], ob, True))
            rf.wait()
            rb.wait()
        s_last = (N_DEV - 1) % SLOTS
        amax = jnp.maximum(amax, gemm(cf[s_last, :, :], nxt, False))
        amax = jnp.maximum(amax, gemm(cb[s_last, :, :], prv, True))

        amax_tx[0, :] = jnp.full((128,), amax, jnp.float32)
        amax_rx[pl.ds(l, 1), :] = amax_tx[0:1, :]
        descs = []
        for j in range(1, N_DEV):
            tgt = (l + j) % N_DEV
            d = pltpu.make_async_remote_copy(
                src_ref=amax_tx.at[pl.ds(0, 1)],
                dst_ref=amax_rx.at[pl.ds(l, 1)],
                send_sem=a_send.at[j], recv_sem=a_recv.at[j],
                device_id=(tgt,), device_id_type=pl.DeviceIdType.MESH)
            d.start()
            descs.append(d)
        for d in descs:
            d.wait()

        scale = jnp.max(amax_rx[...]) / 448.0
        y = out_ref[...]
        q = (y / scale).astype(jnp.float8_e4m3fn).astype(jnp.float32)
        out_ref[...] = q * scale

    return pl.pallas_call(
        body,
        out_shape=jax.ShapeDtypeStruct((m_tot, n_per), jnp.float32),
        in_specs=[pl.BlockSpec(memory_space=pltpu.VMEM),
                  pl.BlockSpec(memory_space=pltpu.VMEM)],
        out_specs=pl.BlockSpec(memory_space=pltpu.VMEM),
        scratch_shapes=[
            pltpu.VMEM((SLOTS, m_half, k), jnp.float32),
            pltpu.VMEM((SLOTS, m_half, k), jnp.float32),
            pltpu.VMEM((1, 128), jnp.float32),
            pltpu.VMEM((N_DEV, 128), jnp.float32),
            pltpu.SemaphoreType.DMA((SLOTS,)),
            pltpu.SemaphoreType.DMA((SLOTS,)),
            pltpu.SemaphoreType.DMA((SLOTS,)),
            pltpu.SemaphoreType.DMA((SLOTS,)),
            pltpu.SemaphoreType.DMA((N_DEV,)),
            pltpu.SemaphoreType.DMA((N_DEV,)),
        ],
    )(x, w_mat)


# device time: 274168 ns/iter; 1.2901x vs baseline; 1.2901x over previous
import jax
import jax.numpy as jnp
from jax import lax
from jax.experimental import pallas as pl
from jax.experimental.pallas import tpu as pltpu

N_DEV = 8
N_HOP = 5


def _ring(q):
    return jnp.where(q < 4, q, 11 - q)


def kernel(x, w_mat):
    m_full, k = x.shape
    m_half = m_full // 2
    n_per = w_mat.shape[1]
    m_tot = N_DEV * m_full

    def body(x_ref, w_ref, out_ref, cf, cb, cc, amax_tx, amax_rx,
             sf_sems, rf_sems, sb_sems, rb_sems, cs_sems, cr_sems,
             a_send, a_recv):
        l = lax.axis_index("i")
        p = _ring(l)
        nxt = _ring((p + 1) % N_DEV)
        prv = _ring((p - 1) % N_DEV)
        is_odd = lax.rem(p, 2) == 1
        chord_l = _ring(jnp.where(is_odd, p - 3, p + 3) % N_DEV)

        cf[0, :, :] = x_ref[:m_half, :]
        cb[0, :, :] = x_ref[m_half:, :]

        def cc_slot(i):
            return cc.at[i]

        n_half = n_per // 2

        def gemm(chunk, row0):
            m = jnp.float32(0)
            for j in range(2):
                wj = w_ref[:, pl.ds(j * n_half, n_half)]
                blk = jnp.dot(chunk, wj, preferred_element_type=jnp.float32,
                              precision=lax.Precision.HIGHEST)
                out_ref[pl.ds(row0, m_half), pl.ds(j * n_half, n_half)] = blk
                m = jnp.maximum(m, jnp.max(jnp.abs(blk)))
            return m

        def frow(pos):
            return _ring(pos % N_DEV) * m_full

        def brow(pos):
            return _ring(pos % N_DEV) * m_full + m_half

        def chord_row(hc):
            delta = 2 if hc % 2 == 1 else 1
            got_F = jnp.logical_not(is_odd) if hc <= 2 else is_odd
            return jnp.where(got_F, frow(p + delta), brow(p - delta))

        amax = jnp.float32(0)
        for h in range(N_HOP):
            s = h % 2
            r = (h + 1) % 2
            rf = pltpu.make_async_remote_copy(
                src_ref=cf.at[s], dst_ref=cf.at[r],
                send_sem=sf_sems.at[s], recv_sem=rf_sems.at[r],
                device_id=(nxt,), device_id_type=pl.DeviceIdType.MESH)
            rb = pltpu.make_async_remote_copy(
                src_ref=cb.at[s], dst_ref=cb.at[r],
                send_sem=sb_sems.at[s], recv_sem=rb_sems.at[r],
                device_id=(prv,), device_id_type=pl.DeviceIdType.MESH)
            rf.start()
            rb.start()
            if h >= 1:
                send_F = is_odd if h <= 2 else jnp.logical_not(is_odd)

                @pl.when(send_F)
                def _():
                    pltpu.make_async_remote_copy(
                        src_ref=cf.at[s], dst_ref=cc_slot(s),
                        send_sem=cs_sems.at[h - 1], recv_sem=cr_sems.at[h - 1],
                        device_id=(chord_l,),
                        device_id_type=pl.DeviceIdType.MESH).start()

                @pl.when(jnp.logical_not(send_F))
                def _():
                    pltpu.make_async_remote_copy(
                        src_ref=cb.at[s], dst_ref=cc_slot(s),
                        send_sem=cs_sems.at[h - 1], recv_sem=cr_sems.at[h - 1],
                        device_id=(chord_l,),
                        device_id_type=pl.DeviceIdType.MESH).start()

            amax = jnp.maximum(amax, gemm(cf[s, :, :], frow(p - h)))
            amax = jnp.maximum(amax, gemm(cb[s, :, :], brow(p + h)))
            if h >= 2:
                amax = jnp.maximum(amax, gemm(cc[r, :, :], chord_row(h - 1)))

            rf.wait()
            rb.wait()
            if h >= 1:
                dw = pltpu.make_async_remote_copy(
                    src_ref=cf.at[s], dst_ref=cc_slot(s),
                    send_sem=cs_sems.at[h - 1], recv_sem=cr_sems.at[h - 1],
                    device_id=(chord_l,),
                    device_id_type=pl.DeviceIdType.MESH)
                dw.wait_send()
                dw.wait_recv()

        amax = jnp.maximum(amax, gemm(cf[N_HOP % 2, :, :], frow(p - N_HOP)))
        amax = jnp.maximum(amax, gemm(cb[N_HOP % 2, :, :], brow(p + N_HOP)))
        amax = jnp.maximum(amax, gemm(cc[(N_HOP - 1) % 2, :, :],
                                      chord_row(N_HOP - 1)))

        amax_tx[0, :] = jnp.full((128,), amax, jnp.float32)
        amax_rx[pl.ds(l, 1), :] = amax_tx[0:1, :]
        descs = []
        for j in range(1, N_DEV):
            tgt = (l + j) % N_DEV
            d = pltpu.make_async_remote_copy(
                src_ref=amax_tx.at[pl.ds(0, 1)],
                dst_ref=amax_rx.at[pl.ds(l, 1)],
                send_sem=a_send.at[j], recv_sem=a_recv.at[j],
                device_id=(tgt,), device_id_type=pl.DeviceIdType.MESH)
            d.start()
            descs.append(d)
        for d in descs:
            d.wait()

        scale = jnp.max(amax_rx[...]) / 448.0
        blk_rows = m_tot // 16
        for i in range(16):
            y = out_ref[pl.ds(i * blk_rows, blk_rows), :]
            q8 = (y / scale).astype(jnp.float8_e4m3fn).astype(jnp.float32)
            out_ref[pl.ds(i * blk_rows, blk_rows), :] = q8 * scale

    return pl.pallas_call(
        body,
        out_shape=jax.ShapeDtypeStruct((m_tot, n_per), jnp.float32),
        in_specs=[pl.BlockSpec(memory_space=pltpu.VMEM),
                  pl.BlockSpec(memory_space=pltpu.VMEM)],
        out_specs=pl.BlockSpec(memory_space=pltpu.VMEM),
        scratch_shapes=[
            pltpu.VMEM((2, m_half, k), jnp.float32),
            pltpu.VMEM((2, m_half, k), jnp.float32),
            pltpu.VMEM((2, m_half, k), jnp.float32),
            pltpu.VMEM((1, 128), jnp.float32),
            pltpu.VMEM((N_DEV, 128), jnp.float32),
            pltpu.SemaphoreType.DMA((2,)),
            pltpu.SemaphoreType.DMA((2,)),
            pltpu.SemaphoreType.DMA((2,)),
            pltpu.SemaphoreType.DMA((2,)),
            pltpu.SemaphoreType.DMA((4,)),
            pltpu.SemaphoreType.DMA((4,)),
            pltpu.SemaphoreType.DMA((N_DEV,)),
            pltpu.SemaphoreType.DMA((N_DEV,)),
        ],
    )(x, w_mat)


# device time: 273285 ns/iter; 1.2942x vs baseline; 1.0032x over previous
import jax
import jax.numpy as jnp
from jax import lax
from jax.experimental import pallas as pl
from jax.experimental.pallas import tpu as pltpu

N_DEV = 8
N_HOP = 5


def _ring(q):
    return jnp.where(q < 4, q, 11 - q)


def kernel(x, w_mat):
    m_full, k = x.shape
    m_half = m_full // 2
    n_per = w_mat.shape[1]
    m_tot = N_DEV * m_full

    def body(x_ref, w_ref, out_ref, cf, cb, cc, amax_tx, amax_rx,
             sf_sems, rf_sems, sb_sems, rb_sems, cs_sems, cr_sems,
             a_send, a_recv):
        l = lax.axis_index("i")
        p = _ring(l)
        nxt = _ring((p + 1) % N_DEV)
        prv = _ring((p - 1) % N_DEV)
        is_odd = lax.rem(p, 2) == 1
        chord_l = _ring(jnp.where(is_odd, p - 3, p + 3) % N_DEV)

        def cc_slot(i):
            return cc.at[i]

        n_half = n_per // 2

        def gemm(chunk, row0):
            m = jnp.float32(0)
            for j in range(2):
                wj = w_ref[:, pl.ds(j * n_half, n_half)]
                blk = jnp.dot(chunk, wj, preferred_element_type=jnp.float32,
                              precision=lax.Precision.HIGHEST)
                out_ref[pl.ds(row0, m_half), pl.ds(j * n_half, n_half)] = blk
                m = jnp.maximum(m, jnp.max(jnp.abs(blk)))
            return m

        def frow(pos):
            return _ring(pos % N_DEV) * m_full

        def brow(pos):
            return _ring(pos % N_DEV) * m_full + m_half

        def chord_row(hc):
            delta = 2 if hc % 2 == 1 else 1
            got_F = jnp.logical_not(is_odd) if hc <= 2 else is_odd
            return jnp.where(got_F, frow(p + delta), brow(p - delta))

        amax = jnp.float32(0)
        for h in range(N_HOP):
            s = (h - 1) % 2
            r = h % 2
            src_f = x_ref.at[pl.ds(0, m_half)] if h == 0 else cf.at[s]
            src_b = x_ref.at[pl.ds(m_half, m_half)] if h == 0 else cb.at[s]
            rf = pltpu.make_async_remote_copy(
                src_ref=src_f, dst_ref=cf.at[r],
                send_sem=sf_sems.at[r], recv_sem=rf_sems.at[r],
                device_id=(nxt,), device_id_type=pl.DeviceIdType.MESH)
            rb = pltpu.make_async_remote_copy(
                src_ref=src_b, dst_ref=cb.at[r],
                send_sem=sb_sems.at[r], recv_sem=rb_sems.at[r],
                device_id=(prv,), device_id_type=pl.DeviceIdType.MESH)
            rf.start()
            rb.start()
            if h >= 1:
                send_F = is_odd if h <= 2 else jnp.logical_not(is_odd)

                @pl.when(send_F)
                def _():
                    pltpu.make_async_remote_copy(
                        src_ref=cf.at[s], dst_ref=cc_slot(s),
                        send_sem=cs_sems.at[h - 1], recv_sem=cr_sems.at[h - 1],
                        device_id=(chord_l,),
                        device_id_type=pl.DeviceIdType.MESH).start()

                @pl.when(jnp.logical_not(send_F))
                def _():
                    pltpu.make_async_remote_copy(
                        src_ref=cb.at[s], dst_ref=cc_slot(s),
                        send_sem=cs_sems.at[h - 1], recv_sem=cr_sems.at[h - 1],
                        device_id=(chord_l,),
                        device_id_type=pl.DeviceIdType.MESH).start()

            if h == 0:
                amax = jnp.maximum(amax, gemm(x_ref[:m_half, :], frow(p)))
                amax = jnp.maximum(amax, gemm(x_ref[m_half:, :], brow(p)))
            else:
                amax = jnp.maximum(amax, gemm(cf[s, :, :], frow(p - h)))
                amax = jnp.maximum(amax, gemm(cb[s, :, :], brow(p + h)))
            if h >= 2:
                amax = jnp.maximum(amax, gemm(cc[r, :, :], chord_row(h - 1)))

            rf.wait()
            rb.wait()
            if h >= 1:
                dw = pltpu.make_async_remote_copy(
                    src_ref=cf.at[s], dst_ref=cc_slot(s),
                    send_sem=cs_sems.at[h - 1], recv_sem=cr_sems.at[h - 1],
                    device_id=(chord_l,),
                    device_id_type=pl.DeviceIdType.MESH)
                dw.wait_send()
                dw.wait_recv()

        amax = jnp.maximum(amax, gemm(cf[(N_HOP - 1) % 2, :, :],
                                      frow(p - N_HOP)))
        amax = jnp.maximum(amax, gemm(cb[(N_HOP - 1) % 2, :, :],
                                      brow(p + N_HOP)))
        amax = jnp.maximum(amax, gemm(cc[(N_HOP - 2) % 2, :, :],
                                      chord_row(N_HOP - 1)))

        amax_tx[0, :] = jnp.full((128,), amax, jnp.float32)
        amax_rx[pl.ds(l, 1), :] = amax_tx[0:1, :]
        descs = []
        for j in range(1, N_DEV):
            tgt = (l + j) % N_DEV
            d = pltpu.make_async_remote_copy(
                src_ref=amax_tx.at[pl.ds(0, 1)],
                dst_ref=amax_rx.at[pl.ds(l, 1)],
                send_sem=a_send.at[j], recv_sem=a_recv.at[j],
                device_id=(tgt,), device_id_type=pl.DeviceIdType.MESH)
            d.start()
            descs.append(d)
        for d in descs:
            d.wait()

        scale = jnp.max(amax_rx[...]) / 448.0
        blk_rows = m_tot // 16
        for i in range(16):
            y = out_ref[pl.ds(i * blk_rows, blk_rows), :]
            q8 = (y / scale).astype(jnp.float8_e4m3fn).astype(jnp.float32)
            out_ref[pl.ds(i * blk_rows, blk_rows), :] = q8 * scale

    return pl.pallas_call(
        body,
        out_shape=jax.ShapeDtypeStruct((m_tot, n_per), jnp.float32),
        in_specs=[pl.BlockSpec(memory_space=pltpu.VMEM),
                  pl.BlockSpec(memory_space=pltpu.VMEM)],
        out_specs=pl.BlockSpec(memory_space=pltpu.VMEM),
        scratch_shapes=[
            pltpu.VMEM((2, m_half, k), jnp.float32),
            pltpu.VMEM((2, m_half, k), jnp.float32),
            pltpu.VMEM((2, m_half, k), jnp.float32),
            pltpu.VMEM((1, 128), jnp.float32),
            pltpu.VMEM((N_DEV, 128), jnp.float32),
            pltpu.SemaphoreType.DMA((2,)),
            pltpu.SemaphoreType.DMA((2,)),
            pltpu.SemaphoreType.DMA((2,)),
            pltpu.SemaphoreType.DMA((2,)),
            pltpu.SemaphoreType.DMA((4,)),
            pltpu.SemaphoreType.DMA((4,)),
            pltpu.SemaphoreType.DMA((N_DEV,)),
            pltpu.SemaphoreType.DMA((N_DEV,)),
        ],
    )(x, w_mat)


# device time: 255762 ns/iter; 1.3829x vs baseline; 1.0685x over previous
import jax
import jax.numpy as jnp
from jax import lax
from jax.experimental import pallas as pl
from jax.experimental.pallas import tpu as pltpu

N_DEV = 8
N_HOP = 5


def _ring(q):
    return jnp.where(q < 4, q, 11 - q)


def kernel(x, w_mat):
    m_full, k = x.shape
    m_half = m_full // 2
    n_per = w_mat.shape[1]
    m_tot = N_DEV * m_full

    def body(x_ref, w_ref, out_ref, cf, cb, cc, amax_tx, amax_rx,
             sf_sems, rf_sems, sb_sems, rb_sems, cs_sems, cr_sems,
             a_send, a_recv):
        l = lax.axis_index("i")
        p = _ring(l)
        nxt = _ring((p + 1) % N_DEV)
        prv = _ring((p - 1) % N_DEV)
        is_odd = lax.rem(p, 2) == 1
        chord_l = _ring(jnp.where(is_odd, p - 3, p + 3) % N_DEV)

        def cc_slot(i):
            return cc.at[i]

        n_half = n_per // 2

        def gemm(chunk, row0):
            m = jnp.float32(0)
            for j in range(2):
                wj = w_ref[:, pl.ds(j * n_half, n_half)]
                blk = jnp.dot(chunk, wj, preferred_element_type=jnp.float32,
                              precision=lax.Precision.DEFAULT)
                out_ref[pl.ds(row0, m_half), pl.ds(j * n_half, n_half)] = blk
                m = jnp.maximum(m, jnp.max(jnp.abs(blk)))
            return m

        def frow(pos):
            return _ring(pos % N_DEV) * m_full

        def brow(pos):
            return _ring(pos % N_DEV) * m_full + m_half

        def chord_row(hc):
            delta = 2 if hc % 2 == 1 else 1
            got_F = jnp.logical_not(is_odd) if hc <= 2 else is_odd
            return jnp.where(got_F, frow(p + delta), brow(p - delta))

        amax = jnp.float32(0)
        for h in range(N_HOP):
            s = (h - 1) % 2
            r = h % 2
            src_f = x_ref.at[pl.ds(0, m_half)] if h == 0 else cf.at[s]
            src_b = x_ref.at[pl.ds(m_half, m_half)] if h == 0 else cb.at[s]
            rf = pltpu.make_async_remote_copy(
                src_ref=src_f, dst_ref=cf.at[r],
                send_sem=sf_sems.at[r], recv_sem=rf_sems.at[r],
                device_id=(nxt,), device_id_type=pl.DeviceIdType.MESH)
            rb = pltpu.make_async_remote_copy(
                src_ref=src_b, dst_ref=cb.at[r],
                send_sem=sb_sems.at[r], recv_sem=rb_sems.at[r],
                device_id=(prv,), device_id_type=pl.DeviceIdType.MESH)
            rf.start()
            rb.start()
            if h >= 1:
                send_F = is_odd if h <= 2 else jnp.logical_not(is_odd)

                @pl.when(send_F)
                def _():
                    pltpu.make_async_remote_copy(
                        src_ref=cf.at[s], dst_ref=cc_slot(s),
                        send_sem=cs_sems.at[h - 1], recv_sem=cr_sems.at[h - 1],
                        device_id=(chord_l,),
                        device_id_type=pl.DeviceIdType.MESH).start()

                @pl.when(jnp.logical_not(send_F))
                def _():
                    pltpu.make_async_remote_copy(
                        src_ref=cb.at[s], dst_ref=cc_slot(s),
                        send_sem=cs_sems.at[h - 1], recv_sem=cr_sems.at[h - 1],
                        device_id=(chord_l,),
                        device_id_type=pl.DeviceIdType.MESH).start()

            if h == 0:
                amax = jnp.maximum(amax, gemm(x_ref[:m_half, :], frow(p)))
                amax = jnp.maximum(amax, gemm(x_ref[m_half:, :], brow(p)))
            else:
                amax = jnp.maximum(amax, gemm(cf[s, :, :], frow(p - h)))
                amax = jnp.maximum(amax, gemm(cb[s, :, :], brow(p + h)))
            if h >= 2:
                amax = jnp.maximum(amax, gemm(cc[r, :, :], chord_row(h - 1)))

            rf.wait()
            rb.wait()
            if h >= 1:
                dw = pltpu.make_async_remote_copy(
                    src_ref=cf.at[s], dst_ref=cc_slot(s),
                    send_sem=cs_sems.at[h - 1], recv_sem=cr_sems.at[h - 1],
                    device_id=(chord_l,),
                    device_id_type=pl.DeviceIdType.MESH)
                dw.wait_send()
                dw.wait_recv()

        amax = jnp.maximum(amax, gemm(cf[(N_HOP - 1) % 2, :, :],
                                      frow(p - N_HOP)))
        amax = jnp.maximum(amax, gemm(cb[(N_HOP - 1) % 2, :, :],
                                      brow(p + N_HOP)))
        amax = jnp.maximum(amax, gemm(cc[(N_HOP - 2) % 2, :, :],
                                      chord_row(N_HOP - 1)))

        amax_tx[0, :] = jnp.full((128,), amax, jnp.float32)
        amax_rx[pl.ds(l, 1), :] = amax_tx[0:1, :]
        descs = []
        for j in range(1, N_DEV):
            tgt = (l + j) % N_DEV
            d = pltpu.make_async_remote_copy(
                src_ref=amax_tx.at[pl.ds(0, 1)],
                dst_ref=amax_rx.at[pl.ds(l, 1)],
                send_sem=a_send.at[j], recv_sem=a_recv.at[j],
                device_id=(tgt,), device_id_type=pl.DeviceIdType.MESH)
            d.start()
            descs.append(d)
        for d in descs:
            d.wait()

        gmax = jnp.max(amax_rx[...])
        scale = gmax / 448.0
        rinv = 448.0 / gmax
        blk_rows = m_tot // 16
        for i in range(16):
            y = out_ref[pl.ds(i * blk_rows, blk_rows), :]
            q8 = (y * rinv).astype(jnp.float8_e4m3fn).astype(jnp.float32)
            out_ref[pl.ds(i * blk_rows, blk_rows), :] = q8 * scale

    return pl.pallas_call(
        body,
        out_shape=jax.ShapeDtypeStruct((m_tot, n_per), jnp.float32),
        in_specs=[pl.BlockSpec(memory_space=pltpu.VMEM),
                  pl.BlockSpec(memory_space=pltpu.VMEM)],
        out_specs=pl.BlockSpec(memory_space=pltpu.VMEM),
        scratch_shapes=[
            pltpu.VMEM((2, m_half, k), jnp.float32),
            pltpu.VMEM((2, m_half, k), jnp.float32),
            pltpu.VMEM((2, m_half, k), jnp.float32),
            pltpu.VMEM((1, 128), jnp.float32),
            pltpu.VMEM((N_DEV, 128), jnp.float32),
            pltpu.SemaphoreType.DMA((2,)),
            pltpu.SemaphoreType.DMA((2,)),
            pltpu.SemaphoreType.DMA((2,)),
            pltpu.SemaphoreType.DMA((2,)),
            pltpu.SemaphoreType.DMA((4,)),
            pltpu.SemaphoreType.DMA((4,)),
            pltpu.SemaphoreType.DMA((N_DEV,)),
            pltpu.SemaphoreType.DMA((N_DEV,)),
        ],
    )(x, w_mat)
